# speedup vs baseline: 6.6759x; 6.6759x over previous
import functools

import jax
import jax.numpy as jnp
from jax import lax
from jax.experimental import pallas as pl
from jax.experimental.pallas import tpu as pltpu
from jax.experimental.pallas import tpu_sc as plsc

D_MODEL = 128
SEQ_LEN = 512
VOCAB = 100
BATCH = 1024

NUM_CORES = 2
NUM_SUBCORES = 16
NUM_WORKERS = NUM_CORES * NUM_SUBCORES
SENT_PER_WORKER = BATCH // NUM_WORKERS
NCHUNK = 4
CHUNK = SEQ_LEN // NCHUNK


def _positional_encoding():
    index = jnp.arange(0, D_MODEL, 2).astype(jnp.float32)
    denominator = jnp.power(10000.0, index / D_MODEL)
    position = jnp.arange(SEQ_LEN, dtype=jnp.float32)[:, None]
    even = jnp.sin(position / denominator)
    odd = jnp.cos(position / denominator)
    return jnp.stack((even, odd), axis=2).reshape(SEQ_LEN, D_MODEL)


def _build_expanded_table(table, pos):
    lblk = 32

    def body(tab_ref, pos_ref, o_ref):
        o_ref[...] = pos_ref[...][:, None, :] + tab_ref[...][None, :, :]

    return pl.pallas_call(
        body,
        grid=(SEQ_LEN // lblk,),
        in_specs=[
            pl.BlockSpec((VOCAB, D_MODEL), lambda i: (0, 0)),
            pl.BlockSpec((lblk, D_MODEL), lambda i: (i, 0)),
        ],
        out_specs=pl.BlockSpec((lblk, VOCAB, D_MODEL), lambda i: (i, 0, 0)),
        out_shape=jax.ShapeDtypeStruct((SEQ_LEN, VOCAB, D_MODEL), jnp.float32),
    )(table, pos)


def _sc_gather(x3, e2, offs):
    mesh = plsc.VectorSubcoreMesh(
        core_axis_name="c", subcore_axis_name="s",
        num_cores=NUM_CORES, num_subcores=NUM_SUBCORES)

    @functools.partial(
        pl.kernel,
        out_type=jax.ShapeDtypeStruct((BATCH, NCHUNK, CHUNK, D_MODEL),
                                      jnp.float32),
        mesh=mesh,
        scratch_types=[
            pltpu.VMEM((NCHUNK, CHUNK), jnp.int32),
            pltpu.VMEM((NCHUNK, CHUNK), jnp.int32),
            pltpu.VMEM((NCHUNK, CHUNK), jnp.int32),
            pltpu.VMEM((NCHUNK, CHUNK, D_MODEL), jnp.float32),
            pltpu.SemaphoreType.DMA,
            pltpu.SemaphoreType.DMA,
            pltpu.SemaphoreType.DMA,
            pltpu.SemaphoreType.DMA,
            pltpu.SemaphoreType.DMA,
            pltpu.SemaphoreType.DMA,
            pltpu.SemaphoreType.DMA,
            pltpu.SemaphoreType.DMA,
        ],
    )
    def k(x_ref, e_ref, offs_ref, out_ref, idx_v, offs_v, comb_v, bufs,
          g0, g1, g2, g3, s0, s1, s2, s3):
        gsems = (g0, g1, g2, g3)
        ssems = (s0, s1, s2, s3)
        wid = lax.axis_index("s") * NUM_CORES + lax.axis_index("c")
        base = wid * SENT_PER_WORKER
        pltpu.sync_copy(offs_ref, offs_v)

        def sentence(b, carry):
            bb = base + b
            pltpu.sync_copy(x_ref.at[bb], idx_v)
            for c in range(NCHUNK):
                for k16 in range(CHUNK // 16):
                    sl = pl.ds(k16 * 16, 16)
                    comb_v[c, sl] = idx_v[c, sl] + offs_v[c, sl]
            ghs = [
                pltpu.async_copy(e_ref.at[comb_v.at[c]], bufs.at[c], gsems[c])
                for c in range(NCHUNK)
            ]
            shs = []
            for c in range(NCHUNK):
                ghs[c].wait()
                shs.append(
                    pltpu.async_copy(bufs.at[c], out_ref.at[bb, c], ssems[c]))
            for h in shs:
                h.wait()
            return carry

        lax.fori_loop(0, SENT_PER_WORKER, sentence, 0)

    return k(x3, e2, offs)


def kernel(x, table):
    pos = _positional_encoding()
    e = _build_expanded_table(table, pos)
    e2 = e.reshape(SEQ_LEN * VOCAB, D_MODEL)
    x3 = x.astype(jnp.int32).reshape(BATCH, NCHUNK, CHUNK)
    offs = (jnp.arange(SEQ_LEN, dtype=jnp.int32) * VOCAB).reshape(NCHUNK, CHUNK)
    out4 = _sc_gather(x3, e2, offs)
    return out4.reshape(BATCH, SEQ_LEN, D_MODEL)


# speedup vs baseline: 7.4545x; 1.1166x over previous
import functools

import jax
import jax.numpy as jnp
from jax import lax
from jax.experimental import pallas as pl
from jax.experimental.pallas import tpu as pltpu
from jax.experimental.pallas import tpu_sc as plsc

D_MODEL = 128
SEQ_LEN = 512
VOCAB = 100
BATCH = 1024

NUM_CORES = 2
NUM_SUBCORES = 16
NUM_WORKERS = NUM_CORES * NUM_SUBCORES
SENT_PER_WORKER = BATCH // NUM_WORKERS
NCHUNK = 4
CHUNK = SEQ_LEN // NCHUNK


def _positional_encoding():
    index = jnp.arange(0, D_MODEL, 2).astype(jnp.float32)
    denominator = jnp.power(10000.0, index / D_MODEL)
    position = jnp.arange(SEQ_LEN, dtype=jnp.float32)[:, None]
    even = jnp.sin(position / denominator)
    odd = jnp.cos(position / denominator)
    return jnp.stack((even, odd), axis=2).reshape(SEQ_LEN, D_MODEL)


def _build_expanded_table(table, pos):
    lblk = 32

    def body(tab_ref, pos_ref, o_ref):
        o_ref[...] = pos_ref[...][:, None, :] + tab_ref[...][None, :, :]

    return pl.pallas_call(
        body,
        grid=(SEQ_LEN // lblk,),
        in_specs=[
            pl.BlockSpec((VOCAB, D_MODEL), lambda i: (0, 0)),
            pl.BlockSpec((lblk, D_MODEL), lambda i: (i, 0)),
        ],
        out_specs=pl.BlockSpec((lblk, VOCAB, D_MODEL), lambda i: (i, 0, 0)),
        out_shape=jax.ShapeDtypeStruct((SEQ_LEN, VOCAB, D_MODEL), jnp.float32),
    )(table, pos)


def _sc_gather(x3, e2, offs):
    mesh = plsc.VectorSubcoreMesh(
        core_axis_name="c", subcore_axis_name="s",
        num_cores=NUM_CORES, num_subcores=NUM_SUBCORES)

    @functools.partial(
        pl.kernel,
        out_type=jax.ShapeDtypeStruct((BATCH, NCHUNK, CHUNK, D_MODEL),
                                      jnp.float32),
        mesh=mesh,
        scratch_types=[
            pltpu.VMEM((2, NCHUNK, CHUNK), jnp.int32),
            pltpu.VMEM((NCHUNK, CHUNK), jnp.int32),
            pltpu.VMEM((2, NCHUNK, CHUNK), jnp.int32),
            pltpu.VMEM((NCHUNK, CHUNK, D_MODEL), jnp.float32),
            pltpu.SemaphoreType.DMA,
            pltpu.SemaphoreType.DMA,
            pltpu.SemaphoreType.DMA,
            pltpu.SemaphoreType.DMA,
            pltpu.SemaphoreType.DMA,
            pltpu.SemaphoreType.DMA,
            pltpu.SemaphoreType.DMA,
            pltpu.SemaphoreType.DMA,
            pltpu.SemaphoreType.DMA,
            pltpu.SemaphoreType.DMA,
        ],
    )
    def k(x_ref, e_ref, offs_ref, out_ref, idx_v, offs_v, comb_v, bufs,
          g0, g1, g2, g3, s0, s1, s2, s3, i0, i1):
        gsems = (g0, g1, g2, g3)
        ssems = (s0, s1, s2, s3)
        isems = (i0, i1)
        wid = lax.axis_index("s") * NUM_CORES + lax.axis_index("c")
        base = wid * SENT_PER_WORKER
        pltpu.sync_copy(offs_ref, offs_v)
        pltpu.async_copy(x_ref.at[base], idx_v.at[0], isems[0])
        pltpu.async_copy(x_ref.at[base + 1], idx_v.at[1], isems[1])

        def pair(p, carry):
            for half in range(2):
                b = 2 * p + half
                bb = base + b
                pltpu.make_async_copy(
                    x_ref.at[bb], idx_v.at[half], isems[half]).wait()
                for c in range(NCHUNK):
                    for k16 in range(CHUNK // 16):
                        sl = pl.ds(k16 * 16, 16)
                        comb_v[half, c, sl] = (
                            idx_v[half, c, sl] + offs_v[c, sl])
                @pl.when(b + 2 < SENT_PER_WORKER)
                def _prefetch():
                    pltpu.async_copy(
                        x_ref.at[bb + 2], idx_v.at[half], isems[half])

                for c in range(NCHUNK):
                    if half == 0:
                        @pl.when(p > 0)
                        def _wait_prev():
                            pltpu.make_async_copy(
                                bufs.at[c], out_ref.at[bb, c],
                                ssems[c]).wait()
                    else:
                        pltpu.make_async_copy(
                            bufs.at[c], out_ref.at[bb, c], ssems[c]).wait()
                    pltpu.async_copy(
                        e_ref.at[comb_v.at[half, c]], bufs.at[c], gsems[c])
                for c in range(NCHUNK):
                    pltpu.make_async_copy(
                        e_ref.at[comb_v.at[half, c]], bufs.at[c],
                        gsems[c]).wait()
                    pltpu.async_copy(bufs.at[c], out_ref.at[bb, c], ssems[c])
            return carry

        lax.fori_loop(0, SENT_PER_WORKER // 2, pair, 0)
        last = base + SENT_PER_WORKER - 1
        for c in range(NCHUNK):
            pltpu.make_async_copy(
                bufs.at[c], out_ref.at[last, c], ssems[c]).wait()

    return k(x3, e2, offs)


def kernel(x, table):
    pos = _positional_encoding()
    e = _build_expanded_table(table, pos)
    e2 = e.reshape(SEQ_LEN * VOCAB, D_MODEL)
    x3 = x.astype(jnp.int32).reshape(BATCH, NCHUNK, CHUNK)
    offs = (jnp.arange(SEQ_LEN, dtype=jnp.int32) * VOCAB).reshape(NCHUNK, CHUNK)
    out4 = _sc_gather(x3, e2, offs)
    return out4.reshape(BATCH, SEQ_LEN, D_MODEL)


# speedup vs baseline: 10.5432x; 1.4143x over previous
import functools

import jax
import jax.numpy as jnp
from jax import lax
from jax.experimental import pallas as pl
from jax.experimental.pallas import tpu as pltpu
from jax.experimental.pallas import tpu_sc as plsc

D_MODEL = 128
SEQ_LEN = 512
VOCAB = 100
BATCH = 1024

NUM_CORES = 2
NUM_SUBCORES = 16
NUM_WORKERS = NUM_CORES * NUM_SUBCORES
SENT_PER_WORKER = BATCH // NUM_WORKERS
NCHUNK = 4
CHUNK = SEQ_LEN // NCHUNK


def _positional_encoding():
    index = jnp.arange(0, D_MODEL, 2).astype(jnp.float32)
    denominator = jnp.power(10000.0, index / D_MODEL)
    position = jnp.arange(SEQ_LEN, dtype=jnp.float32)[:, None]
    even = jnp.sin(position / denominator)
    odd = jnp.cos(position / denominator)
    return jnp.stack((even, odd), axis=2).reshape(SEQ_LEN, D_MODEL)


def _build_expanded_table(table, pos):
    lblk = 32

    def body(tab_ref, pos_ref, o_ref):
        o_ref[...] = pos_ref[...][:, None, :] + tab_ref[...][None, :, :]

    return pl.pallas_call(
        body,
        grid=(SEQ_LEN // lblk,),
        in_specs=[
            pl.BlockSpec((VOCAB, D_MODEL), lambda i: (0, 0)),
            pl.BlockSpec((lblk, D_MODEL), lambda i: (i, 0)),
        ],
        out_specs=pl.BlockSpec((lblk, VOCAB, D_MODEL), lambda i: (i, 0, 0)),
        out_shape=jax.ShapeDtypeStruct((SEQ_LEN, VOCAB, D_MODEL), jnp.float32),
    )(table, pos)


NPHASE = 16
PCH = SEQ_LEN // NPHASE
ESP_ROWS = PCH * VOCAB
NBUF = 4


def _sc_gather(x2, e2, offs):
    mesh = plsc.VectorSubcoreMesh(
        core_axis_name="c", subcore_axis_name="s",
        num_cores=NUM_CORES, num_subcores=NUM_SUBCORES)

    @functools.partial(
        pl.kernel,
        out_type=jax.ShapeDtypeStruct((BATCH, NPHASE, PCH, D_MODEL),
                                      jnp.float32),
        mesh=mesh,
        scratch_types=[
            pltpu.VMEM((SENT_PER_WORKER, SEQ_LEN), jnp.int32),
            pltpu.VMEM((SENT_PER_WORKER, SEQ_LEN), jnp.int32),
            pltpu.VMEM((SEQ_LEN,), jnp.int32),
            pltpu.VMEM((NBUF, PCH, D_MODEL), jnp.float32),
            pltpu.VMEM_SHARED((ESP_ROWS, D_MODEL), jnp.float32),
            pltpu.VMEM_SHARED((ESP_ROWS, D_MODEL), jnp.float32),
            pltpu.SemaphoreType.DMA,
            pltpu.SemaphoreType.DMA,
            pltpu.SemaphoreType.DMA,
            pltpu.SemaphoreType.DMA,
            pltpu.SemaphoreType.DMA,
            pltpu.SemaphoreType.DMA,
            pltpu.SemaphoreType.DMA,
            pltpu.SemaphoreType.DMA,
            pltpu.SemaphoreType.DMA,
            pltpu.SemaphoreType.DMA,
        ],
    )
    def k(x_ref, e_ref, offs_ref, out_ref, idx_all, comb_all, offs_v, bufs,
          e_sp0, e_sp1, g0, g1, g2, g3, s0, s1, s2, s3, t0, t1):
        gsems = (g0, g1, g2, g3)
        ssems = (s0, s1, s2, s3)
        stgsems = (t0, t1)
        e_sps = (e_sp0, e_sp1)
        sid = lax.axis_index("s")
        wid = sid * NUM_CORES + lax.axis_index("c")
        base = wid * SENT_PER_WORKER

        @pl.when(sid == 0)
        def _stage01():
            pltpu.async_copy(
                e_ref.at[pl.ds(0, ESP_ROWS)], e_sps[0], stgsems[0])
            pltpu.async_copy(
                e_ref.at[pl.ds(ESP_ROWS, ESP_ROWS)], e_sps[1], stgsems[1])

        pltpu.sync_copy(offs_ref, offs_v)
        pltpu.sync_copy(x_ref.at[pl.ds(base, SENT_PER_WORKER)], idx_all)

        def combi(b, carry):
            for r in range(SEQ_LEN // 16):
                sl = pl.ds(r * 16, 16)
                comb_all[b, sl] = idx_all[b, sl] + offs_v[sl]
            return carry

        lax.fori_loop(0, SENT_PER_WORKER, combi, 0)

        for ph in range(NPHASE):
            e_sp = e_sps[ph % 2]

            @pl.when(sid == 0)
            def _wait_stage():
                pltpu.make_async_copy(
                    e_ref.at[pl.ds(ph * ESP_ROWS, ESP_ROWS)], e_sp,
                    stgsems[ph % 2]).wait()

            plsc.subcore_barrier()

            def group(g, carry):
                for j in range(NBUF):
                    b = g * NBUF + j
                    bb = base + b
                    if ph == 0:
                        @pl.when(g > 0)
                        def _wait_prev():
                            pltpu.make_async_copy(
                                bufs.at[j], out_ref.at[bb, ph],
                                ssems[j]).wait()
                    else:
                        pltpu.make_async_copy(
                            bufs.at[j], out_ref.at[bb, ph], ssems[j]).wait()
                    pltpu.async_copy(
                        e_sp.at[comb_all.at[b, pl.ds(ph * PCH, PCH)]],
                        bufs.at[j], gsems[j])
                for j in range(NBUF):
                    b = g * NBUF + j
                    bb = base + b
                    pltpu.make_async_copy(
                        e_sp.at[comb_all.at[b, pl.ds(ph * PCH, PCH)]],
                        bufs.at[j], gsems[j]).wait()
                    pltpu.async_copy(bufs.at[j], out_ref.at[bb, ph],
                                     ssems[j])
                return carry

            lax.fori_loop(0, SENT_PER_WORKER // NBUF, group, 0)

            plsc.subcore_barrier()
            if ph + 2 < NPHASE:
                @pl.when(sid == 0)
                def _stage_next():
                    pltpu.async_copy(
                        e_ref.at[pl.ds((ph + 2) * ESP_ROWS, ESP_ROWS)],
                        e_sps[ph % 2], stgsems[ph % 2])

        for j in range(NBUF):
            bb = base + SENT_PER_WORKER - NBUF + j
            pltpu.make_async_copy(
                bufs.at[j], out_ref.at[bb, NPHASE - 1], ssems[j]).wait()

    return k(x2, e2, offs)


def kernel(x, table):
    pos = _positional_encoding()
    e = _build_expanded_table(table, pos)
    e2 = e.reshape(SEQ_LEN * VOCAB, D_MODEL)
    x2 = x.astype(jnp.int32)
    offs = (jnp.arange(SEQ_LEN, dtype=jnp.int32) % PCH) * VOCAB
    out4 = _sc_gather(x2, e2, offs)
    return out4.reshape(BATCH, SEQ_LEN, D_MODEL)


# speedup vs baseline: 12.4089x; 1.1770x over previous
import functools

import jax
import jax.numpy as jnp
from jax import lax
from jax.experimental import pallas as pl
from jax.experimental.pallas import tpu as pltpu
from jax.experimental.pallas import tpu_sc as plsc

D_MODEL = 128
SEQ_LEN = 512
VOCAB = 100
BATCH = 1024

NUM_CORES = 2
NUM_SUBCORES = 16
NUM_WORKERS = NUM_CORES * NUM_SUBCORES
SENT_PER_WORKER = BATCH // NUM_WORKERS
NCHUNK = 4
CHUNK = SEQ_LEN // NCHUNK


def _positional_encoding():
    index = jnp.arange(0, D_MODEL, 2).astype(jnp.float32)
    denominator = jnp.power(10000.0, index / D_MODEL)
    position = jnp.arange(SEQ_LEN, dtype=jnp.float32)[:, None]
    even = jnp.sin(position / denominator)
    odd = jnp.cos(position / denominator)
    return jnp.stack((even, odd), axis=2).reshape(SEQ_LEN, D_MODEL)


VPAD = 104


def _build_expanded_table(table_pad, pos):
    lblk = 32

    def body(tab_ref, pos_ref, o_ref):
        o_ref[...] = (
            pos_ref[...][:, None, :] + tab_ref[...][None, :, :]
        ).reshape(lblk * VPAD, D_MODEL)

    return pl.pallas_call(
        body,
        grid=(SEQ_LEN // lblk,),
        in_specs=[
            pl.BlockSpec((VPAD, D_MODEL), lambda i: (0, 0)),
            pl.BlockSpec((lblk, D_MODEL), lambda i: (i, 0)),
        ],
        out_specs=pl.BlockSpec((lblk * VPAD, D_MODEL), lambda i: (i, 0)),
        out_shape=jax.ShapeDtypeStruct((SEQ_LEN * VPAD, D_MODEL),
                                       jnp.float32),
    )(table_pad, pos)


NPHASE = 16
PCH = SEQ_LEN // NPHASE
ESP_ROWS = PCH * VPAD
NBUF = 8


def _sc_gather(x2, e2, offs):
    mesh = plsc.VectorSubcoreMesh(
        core_axis_name="c", subcore_axis_name="s",
        num_cores=NUM_CORES, num_subcores=NUM_SUBCORES)

    @functools.partial(
        pl.kernel,
        out_type=jax.ShapeDtypeStruct((BATCH, NPHASE, PCH, D_MODEL),
                                      jnp.float32),
        mesh=mesh,
        scratch_types=[
            pltpu.VMEM((SENT_PER_WORKER, SEQ_LEN), jnp.int32),
            pltpu.VMEM((SENT_PER_WORKER, SEQ_LEN), jnp.int32),
            pltpu.VMEM((SEQ_LEN,), jnp.int32),
            pltpu.VMEM((NBUF, PCH, D_MODEL), jnp.float32),
            pltpu.VMEM_SHARED((ESP_ROWS, D_MODEL), jnp.float32),
            pltpu.VMEM_SHARED((ESP_ROWS, D_MODEL), jnp.float32),
        ] + [pltpu.SemaphoreType.DMA] * (2 * NBUF + 2),
    )
    def k(x_ref, e_ref, offs_ref, out_ref, idx_all, comb_all, offs_v, bufs,
          e_sp0, e_sp1, *sems):
        gsems = sems[:NBUF]
        ssems = sems[NBUF:2 * NBUF]
        stgsems = sems[2 * NBUF:]
        e_sps = (e_sp0, e_sp1)
        sid = lax.axis_index("s")
        wid = sid * NUM_CORES + lax.axis_index("c")
        base = wid * SENT_PER_WORKER

        @pl.when(sid == 0)
        def _stage01():
            pltpu.async_copy(
                e_ref.at[pl.ds(0, ESP_ROWS)], e_sps[0], stgsems[0])
            pltpu.async_copy(
                e_ref.at[pl.ds(ESP_ROWS, ESP_ROWS)], e_sps[1], stgsems[1])

        pltpu.sync_copy(offs_ref, offs_v)
        pltpu.sync_copy(x_ref.at[pl.ds(base, SENT_PER_WORKER)], idx_all)

        def combi(b, carry):
            for r in range(SEQ_LEN // 16):
                sl = pl.ds(r * 16, 16)
                comb_all[b, sl] = idx_all[b, sl] + offs_v[sl]
            return carry

        lax.fori_loop(0, SENT_PER_WORKER, combi, 0)

        for ph in range(NPHASE):
            e_sp = e_sps[ph % 2]

            @pl.when(sid == 0)
            def _wait_stage():
                pltpu.make_async_copy(
                    e_ref.at[pl.ds(ph * ESP_ROWS, ESP_ROWS)], e_sp,
                    stgsems[ph % 2]).wait()

            plsc.subcore_barrier()

            def group(g, carry):
                for j in range(NBUF):
                    b = g * NBUF + j
                    bb = base + b
                    if ph == 0:
                        @pl.when(g > 0)
                        def _wait_prev():
                            pltpu.make_async_copy(
                                bufs.at[j], out_ref.at[bb, ph],
                                ssems[j]).wait()
                    else:
                        pltpu.make_async_copy(
                            bufs.at[j], out_ref.at[bb, ph], ssems[j]).wait()
                    pltpu.async_copy(
                        e_sp.at[comb_all.at[b, pl.ds(ph * PCH, PCH)]],
                        bufs.at[j], gsems[j])
                for j in range(NBUF):
                    b = g * NBUF + j
                    bb = base + b
                    pltpu.make_async_copy(
                        e_sp.at[comb_all.at[b, pl.ds(ph * PCH, PCH)]],
                        bufs.at[j], gsems[j]).wait()
                    pltpu.async_copy(bufs.at[j], out_ref.at[bb, ph],
                                     ssems[j])
                return carry

            lax.fori_loop(0, SENT_PER_WORKER // NBUF, group, 0)

            plsc.subcore_barrier()
            if ph + 2 < NPHASE:
                @pl.when(sid == 0)
                def _stage_next():
                    pltpu.async_copy(
                        e_ref.at[pl.ds((ph + 2) * ESP_ROWS, ESP_ROWS)],
                        e_sps[ph % 2], stgsems[ph % 2])

        for j in range(NBUF):
            bb = base + SENT_PER_WORKER - NBUF + j
            pltpu.make_async_copy(
                bufs.at[j], out_ref.at[bb, NPHASE - 1], ssems[j]).wait()

    return k(x2, e2, offs)


def kernel(x, table):
    pos = _positional_encoding()
    table_pad = jnp.pad(table, ((0, VPAD - VOCAB), (0, 0)))
    e2 = _build_expanded_table(table_pad, pos)
    x2 = x.astype(jnp.int32)
    offs = (jnp.arange(SEQ_LEN, dtype=jnp.int32) % PCH) * VPAD
    out4 = _sc_gather(x2, e2, offs)
    return out4.reshape(BATCH, SEQ_LEN, D_MODEL)
